# Initial kernel scaffold; baseline (speedup 1.0000x reference)
import jax
import jax.numpy as jnp
from jax.experimental import pallas as pl


def kernel(sample):
    raise NotImplementedError("write your pallas kernel here")


# speedup vs baseline: 6.5102x; 6.5102x over previous
import jax
import jax.numpy as jnp
from jax.experimental import pallas as pl


_BT = 16384


def _mix_body(s_ref, o_ref):
    s = s_ref[...]
    o_ref[:, 0, :] = s[:, 0, :] + s[:, 1, :] + s[:, 2, :]
    o_ref[:, 1, :] = s[:, 3, :]


def kernel(sample):
    B, C, T = sample.shape
    grid = (T // _BT,)
    return pl.pallas_call(
        _mix_body,
        grid=grid,
        in_specs=[pl.BlockSpec((B, C, _BT), lambda i: (0, 0, i))],
        out_specs=pl.BlockSpec((B, 2, _BT), lambda i: (0, 0, i)),
        out_shape=jax.ShapeDtypeStruct((B, 2, T), sample.dtype),
    )(sample)


# speedup vs baseline: 8.3890x; 1.2886x over previous
import jax
import jax.numpy as jnp
from jax.experimental import pallas as pl


_BT = 65536


def _mix_body(s_ref, o_ref):
    s = s_ref[...]
    o_ref[:, 0, :] = s[:, 0, :] + s[:, 1, :] + s[:, 2, :]
    o_ref[:, 1, :] = s[:, 3, :]


def kernel(sample):
    B, C, T = sample.shape
    grid = (T // _BT,)
    return pl.pallas_call(
        _mix_body,
        grid=grid,
        in_specs=[pl.BlockSpec((B, C, _BT), lambda i: (0, 0, i))],
        out_specs=pl.BlockSpec((B, 2, _BT), lambda i: (0, 0, i)),
        out_shape=jax.ShapeDtypeStruct((B, 2, T), sample.dtype),
    )(sample)


# speedup vs baseline: 8.7086x; 1.0381x over previous
import jax
import jax.numpy as jnp
from jax.experimental import pallas as pl


_BT = 131072


def _mix_body(s_ref, o_ref):
    s = s_ref[...]
    o_ref[:, 0, :] = s[:, 0, :] + s[:, 1, :] + s[:, 2, :]
    o_ref[:, 1, :] = s[:, 3, :]


def kernel(sample):
    B, C, T = sample.shape
    grid = (T // _BT,)
    return pl.pallas_call(
        _mix_body,
        grid=grid,
        in_specs=[pl.BlockSpec((B, C, _BT), lambda i: (0, 0, i))],
        out_specs=pl.BlockSpec((B, 2, _BT), lambda i: (0, 0, i)),
        out_shape=jax.ShapeDtypeStruct((B, 2, T), sample.dtype),
    )(sample)
